# Initial kernel scaffold; baseline (speedup 1.0000x reference)
import jax
import jax.numpy as jnp
from jax.experimental import pallas as pl


def kernel(xyz_b3n, W_stem, b_stem, W1, g1, bt1, W2, g2, bt2, W3, g3, bt3, W4, g4, bt4):
    raise NotImplementedError("write your pallas kernel here")


# speedup vs baseline: 4.3258x; 4.3258x over previous
import functools

import jax
import jax.numpy as jnp
from jax.experimental import pallas as pl

K_NN = 16
_EPS = 1e-5


def _mm_body(w_ref, x_ref, b_ref, o_ref):
    x = x_ref[0]
    o_ref[0] = jax.lax.dot_general(
        w_ref[...], x, (((1,), (0,)), ((), ())),
        preferred_element_type=jnp.float32, precision=jax.lax.Precision.HIGHEST) + b_ref[...]


def _mm(W, X, b=None):
    B, C, N = X.shape
    O = W.shape[0]
    if b is None:
        b = jnp.zeros((O,), jnp.float32)
    return pl.pallas_call(
        _mm_body,
        grid=(B,),
        in_specs=[pl.BlockSpec((O, C), lambda bb: (0, 0)),
                  pl.BlockSpec((1, C, N), lambda bb: (bb, 0, 0)),
                  pl.BlockSpec((O, 1), lambda bb: (0, 0))],
        out_specs=pl.BlockSpec((1, O, N), lambda bb: (bb, 0, 0)),
        out_shape=jax.ShapeDtypeStruct((B, O, N), jnp.float32),
    )(W, X, b.reshape(O, 1))


def _mmT_body(w_ref, x_ref, b_ref, o_ref):
    x = x_ref[0]
    o_ref[0] = jax.lax.dot_general(
        x, w_ref[...], (((0,), (1,)), ((), ())),
        preferred_element_type=jnp.float32, precision=jax.lax.Precision.HIGHEST) + b_ref[...]


def _mmT(W, X, b=None):
    B, C, N = X.shape
    O = W.shape[0]
    if b is None:
        b = jnp.zeros((O,), jnp.float32)
    return pl.pallas_call(
        _mmT_body,
        grid=(B,),
        in_specs=[pl.BlockSpec((O, C), lambda bb: (0, 0)),
                  pl.BlockSpec((1, C, N), lambda bb: (bb, 0, 0)),
                  pl.BlockSpec((1, O), lambda bb: (0, 0))],
        out_specs=pl.BlockSpec((1, N, O), lambda bb: (bb, 0, 0)),
        out_shape=jax.ShapeDtypeStruct((B, N, O), jnp.float32),
    )(W, X, b.reshape(1, O))


def _edge_knn_body(num_g, k_nn, np_full, *refs):
    pq_ref, pk_ref = refs[0], refs[1]
    g_refs = refs[2:2 + num_g]
    out_refs = refs[2 + num_g:]

    qT = jnp.transpose(pq_ref[0])
    p = pk_ref[0]
    qq = jnp.sum(qT * qT, axis=1, keepdims=True)
    pp = jnp.sum(p * p, axis=0, keepdims=True)
    qp = jax.lax.dot_general(qT.astype(jnp.bfloat16), p.astype(jnp.bfloat16),
                             (((1,), (0,)), ((), ())),
                             preferred_element_type=jnp.float32)
    d0 = qq + pp - 2.0 * qp
    bq = d0.shape[0]
    iota = jax.lax.broadcasted_iota(jnp.int32, (bq, np_full), 1)
    Gs = [g_refs[i][0] for i in range(num_g)]
    inf = jnp.float32(jnp.inf)

    def step(_, carry):
        d = carry[0]
        accs = carry[1:]
        dmin = jnp.min(d, axis=1, keepdims=True)
        sel = jnp.min(jnp.where(d == dmin, iota, np_full), axis=1,
                      keepdims=True)
        ohm = iota == sel
        oh = ohm.astype(jnp.float32)
        new_accs = []
        for gi in range(num_g):
            gv = jax.lax.dot_general(oh, Gs[gi], (((1,), (1,)), ((), ())),
                                     preferred_element_type=jnp.float32, precision=jax.lax.Precision.HIGHEST)
            vmax, vmin, vsum, vsq = accs[4 * gi:4 * gi + 4]
            new_accs += [jnp.maximum(vmax, gv), jnp.minimum(vmin, gv),
                         vsum + gv, vsq + gv * gv]
        return (jnp.where(ohm, inf, d),) + tuple(new_accs)

    init = (d0,)
    for gi in range(num_g):
        O = Gs[gi].shape[0]
        init += (jnp.full((bq, O), -inf, jnp.float32),
                 jnp.full((bq, O), inf, jnp.float32),
                 jnp.zeros((bq, O), jnp.float32),
                 jnp.zeros((bq, O), jnp.float32))
    fin = jax.lax.fori_loop(0, k_nn, step, init)
    for j in range(4 * num_g):
        out_refs[j][0] = fin[1 + j]


def _edge_knn(Pq, Pk, Gs):
    B, _, Nq = Pq.shape
    Np = Pk.shape[2]
    bq = min(256, Nq)
    num_g = len(Gs)
    in_specs = [pl.BlockSpec((1, 3, bq), lambda b, i: (b, 0, i)),
                pl.BlockSpec((1, 3, Np), lambda b, i: (b, 0, 0))]
    out_shapes, out_specs = [], []
    for G in Gs:
        O = G.shape[1]
        in_specs.append(pl.BlockSpec((1, O, Np), lambda b, i: (b, 0, 0)))
        for _ in range(4):
            out_shapes.append(jax.ShapeDtypeStruct((B, Nq, O), jnp.float32))
            out_specs.append(pl.BlockSpec((1, bq, O), lambda b, i: (b, i, 0)))
    body = functools.partial(_edge_knn_body, num_g, K_NN, Np)
    outs = pl.pallas_call(
        body,
        grid=(B, Nq // bq),
        in_specs=in_specs,
        out_specs=out_specs,
        out_shape=out_shapes,
    )(Pq, Pk, *Gs)
    return [tuple(outs[4 * i:4 * i + 4]) for i in range(num_g)]


def _edge_fin_body(k_nn, num_groups, vmax_ref, vmin_ref, vsum_ref, vsq_ref,
                   t_ref, gm_ref, bt_ref, o_ref):
    vmax = vmax_ref[0]
    vmin = vmin_ref[0]
    vsum = vsum_ref[0]
    vsq = vsq_ref[0]
    t = t_ref[0]
    kf = jnp.float32(k_nn)
    sum_v = vsum + kf * t
    sq_v = vsq + 2.0 * t * vsum + kf * t * t
    S = jnp.sum(sum_v, axis=0, keepdims=True)
    Q = jnp.sum(sq_v, axis=0, keepdims=True)
    O = vmax.shape[1]
    Nq = vmax.shape[0]
    cg = O // num_groups
    gi = jax.lax.broadcasted_iota(jnp.int32, (O, O), 0) // cg
    gj = jax.lax.broadcasted_iota(jnp.int32, (O, O), 1) // cg
    Mm = (gi == gj).astype(jnp.float32)
    count = jnp.float32(cg * Nq * k_nn)
    Sg = jax.lax.dot_general(S, Mm, (((1,), (0,)), ((), ())),
                             preferred_element_type=jnp.float32, precision=jax.lax.Precision.HIGHEST)
    Qg = jax.lax.dot_general(Q, Mm, (((1,), (0,)), ((), ())),
                             preferred_element_type=jnp.float32, precision=jax.lax.Precision.HIGHEST)
    mean = Sg / count
    var = Qg / count - mean * mean
    rstd = 1.0 / jnp.sqrt(var + _EPS)
    s = gm_ref[...] * rstd
    c = bt_ref[...] - mean * s
    vsel = jnp.where(s >= 0.0, vmax + t, vmin + t)
    out = jnp.maximum(s * vsel + c, 0.0)
    o_ref[0] = jnp.transpose(out)


def _edge_fin(stats, t, gamma, beta):
    vmax, vmin, vsum, vsq = stats
    B, Nq, O = vmax.shape
    ng = min(8, O)
    body = functools.partial(_edge_fin_body, K_NN, ng)
    spec = pl.BlockSpec((1, Nq, O), lambda b: (b, 0, 0))
    vspec = pl.BlockSpec((1, O), lambda b: (0, 0))
    return pl.pallas_call(
        body,
        grid=(B,),
        in_specs=[spec, spec, spec, spec, spec, vspec, vspec],
        out_specs=pl.BlockSpec((1, O, Nq), lambda b: (b, 0, 0)),
        out_shape=jax.ShapeDtypeStruct((B, O, Nq), jnp.float32),
    )(vmax, vmin, vsum, vsq, t, gamma.reshape(1, O), beta.reshape(1, O))


def _fps_body(m, n_full, p_ref, idx_ref):
    Px = p_ref[:, 0, :]
    Py = p_ref[:, 1, :]
    Pz = p_ref[:, 2, :]
    B = Px.shape[0]
    iN = jax.lax.broadcasted_iota(jnp.int32, (B, n_full), 1)
    im = jax.lax.broadcasted_iota(jnp.int32, (B, m), 1)
    dx = Px - Px[:, 0:1]
    dy = Py - Py[:, 0:1]
    dz = Pz - Pz[:, 0:1]
    d0 = dx * dx + dy * dy + dz * dz
    acc0 = jnp.zeros((B, m), jnp.int32)

    def step(i, carry):
        d, acc = carry
        dmax = jnp.max(d, axis=1, keepdims=True)
        sel = jnp.min(jnp.where(d == dmax, iN, n_full), axis=1,
                      keepdims=True)
        msk = iN == sel
        lx = jnp.sum(jnp.where(msk, Px, 0.0), axis=1, keepdims=True)
        ly = jnp.sum(jnp.where(msk, Py, 0.0), axis=1, keepdims=True)
        lz = jnp.sum(jnp.where(msk, Pz, 0.0), axis=1, keepdims=True)
        ex = Px - lx
        ey = Py - ly
        ez = Pz - lz
        dn = ex * ex + ey * ey + ez * ez
        return jnp.minimum(d, dn), jnp.where(im == i, sel, acc)

    _, acc = jax.lax.fori_loop(1, m, step, (d0, acc0))
    idx_ref[...] = acc


def _fps(P, m):
    B, _, N = P.shape
    return pl.pallas_call(
        functools.partial(_fps_body, m, N),
        out_shape=jax.ShapeDtypeStruct((B, m), jnp.int32),
    )(P)


def _gather_body(n_full, src_ref, idx_ref, o_ref):
    src = src_ref[0]
    col = jnp.transpose(idx_ref[0])
    mb = col.shape[0]
    iota = jax.lax.broadcasted_iota(jnp.int32, (mb, n_full), 1)
    oh = (iota == col).astype(jnp.float32)
    o_ref[0] = jax.lax.dot_general(src, oh, (((1,), (1,)), ((), ())),
                                   preferred_element_type=jnp.float32, precision=jax.lax.Precision.HIGHEST)


def _gather(src, idx):
    B, C, N = src.shape
    m = idx.shape[1]
    mb = min(256, m)
    idx3 = idx.reshape(B, 1, m)
    return pl.pallas_call(
        functools.partial(_gather_body, N),
        grid=(B, m // mb),
        in_specs=[pl.BlockSpec((1, C, N), lambda b, i: (b, 0, 0)),
                  pl.BlockSpec((1, 1, mb), lambda b, i: (b, 0, i))],
        out_specs=pl.BlockSpec((1, C, mb), lambda b, i: (b, 0, i)),
        out_shape=jax.ShapeDtypeStruct((B, C, m), jnp.float32),
    )(src, idx3)


def kernel(xyz_b3n, W_stem, b_stem, W1, g1, bt1, W2, g2, bt2, W3, g3, bt3,
           W4, g4, bt4):
    B, _, N0 = xyz_b3n.shape
    N1 = max(1, int(N0 * 0.25))
    N2 = max(1, int(N1 * 0.25))

    C1 = W1.shape[1] // 2
    W1a = W1[:, :C1]
    W1d = W1[:, C1:] - W1a
    G1 = _mm(W1a @ W_stem, xyz_b3n, W1a @ b_stem)
    t1 = _mmT(W1d @ W_stem, xyz_b3n, W1d @ b_stem)
    (st1,) = _edge_knn(xyz_b3n, xyz_b3n, [G1])
    F0a = _edge_fin(st1, t1, g1, bt1)

    idx1 = _fps(xyz_b3n, N1)
    gat1 = _gather(jnp.concatenate([xyz_b3n, F0a], axis=1), idx1)
    P1 = gat1[:, :3]
    F1_sk = gat1[:, 3:]

    C2 = W2.shape[1] // 2
    W2a = W2[:, :C2]
    W2d = W2[:, C2:] - W2a
    G2 = _mm(W2a, F0a)
    t2 = _mmT(W2d, F1_sk)
    (st2,) = _edge_knn(P1, xyz_b3n, [G2])
    F1a = _edge_fin(st2, t2, g2, bt2)

    idx2 = _fps(P1, N2)
    gat2 = _gather(jnp.concatenate([P1, F1a], axis=1), idx2)
    P2 = gat2[:, :3]
    F2_sk = gat2[:, 3:]

    C3 = W3.shape[1] // 2
    W3a = W3[:, :C3]
    W3d = W3[:, C3:] - W3a
    C4 = W4.shape[1] // 2
    W4a = W4[:, :C4]
    W4d = W4[:, C4:] - W4a
    G3 = _mm(W3a, F1a)
    G4 = _mm(W4a, F1a)
    t3 = _mmT(W3d, F2_sk)
    st3, st4 = _edge_knn(P2, P1, [G3, G4])
    F2_mid = _edge_fin(st3, t3, g3, bt3)
    t4 = _mmT(W4d, F2_mid)
    F2a = _edge_fin(st4, t4, g4, bt4)

    return (xyz_b3n, F0a, P1, F1a, idx1, P2, F2a, idx2)
